# baseline (speedup 1.0000x reference)
import functools

import jax
import jax.numpy as jnp
from jax import lax
from jax.experimental import pallas as pl
from jax.experimental.pallas import tpu as pltpu
from jax.experimental.pallas import tpu_sc as plsc

_N = 10000
_E = 320000
_F = 128
_HF = 64
_H = 3
_NEG = 0.2

_NW = 32
_EPW = _E // _NW
_CB = 2000
_CC = 80
_NCH = _EPW // _CC
_OWN = 624
_TAIL = _N - 16 * _OWN

_NBUF = 4

_BN = 1000


def _proj_body(x_ref, wt_ref, a8_ref, elr_ref, *h_refs):
    h_all = jnp.dot(x_ref[...], wt_ref[...], preferred_element_type=jnp.float32)
    for i in range(2 * _H):
        h_refs[i][...] = h_all[:, i * _HF:(i + 1) * _HF].astype(jnp.bfloat16)
    elr_ref[...] = jnp.dot(h_all, a8_ref[...], preferred_element_type=jnp.float32)


def _project(x, wt, a8):
    hspec = pl.BlockSpec((_BN, _HF), lambda i: (i, 0))
    hshape = jax.ShapeDtypeStruct((_N, _HF), jnp.bfloat16)
    return pl.pallas_call(
        _proj_body,
        grid=(_N // _BN,),
        in_specs=[
            pl.BlockSpec((_BN, _F), lambda i: (i, 0)),
            pl.BlockSpec((_F, _H * _F), lambda i: (0, 0)),
            pl.BlockSpec((_H * _F, 8), lambda i: (0, 0)),
        ],
        out_specs=(pl.BlockSpec((_BN, 8), lambda i: (i, 0)),) + (hspec,) * (2 * _H),
        out_shape=(jax.ShapeDtypeStruct((_N, 8), jnp.float32),) + (hshape,) * (2 * _H),
    )(x, wt, a8)


def _edge_logits_body(src_hbm, dst_hbm, el_hbm, er_hbm,
                      exp_hbm, psum_hbm,
                      el_v, er_v, psum_v, src_c, dst_c, exp_c):
    cid = lax.axis_index("c")
    sid = lax.axis_index("s")
    wid = cid * 16 + sid
    base = pl.multiple_of(wid * _EPW, 8)

    pltpu.sync_copy(el_hbm, el_v)
    pltpu.sync_copy(er_hbm, er_v)

    zeros16 = jnp.zeros((16,), jnp.float32)

    def zbody(i, carry):
        psum_v[pl.ds(i * 16, 16)] = zeros16
        return carry

    lax.fori_loop(0, (_H * _N) // 16, zbody, 0)

    def chunk_body(c, carry):
        off = pl.multiple_of(base + c * _CB, 8)
        pltpu.sync_copy(src_hbm.at[pl.ds(off, _CB)], src_c)
        pltpu.sync_copy(dst_hbm.at[pl.ds(off, _CB)], dst_c)

        def vec_body(j, inner):
            s16 = src_c[pl.ds(j * 16, 16)]
            d16 = dst_c[pl.ds(j * 16, 16)]
            s3 = s16 * 3
            d3 = d16 * 3
            for h in range(_H):
                e = (plsc.load_gather(el_v, [s3 + h])
                     + plsc.load_gather(er_v, [d3 + h]))
                e = jnp.where(e >= 0, e, _NEG * e)
                ex = jnp.exp(e)
                exp_c[pl.ds(h * _CB + j * 16, 16)] = ex
                plsc.addupdate_scatter(psum_v, [d16 + h * _N], ex)
            return inner

        lax.fori_loop(0, _CB // 16, vec_body, 0)
        for h in range(_H):
            pltpu.sync_copy(exp_c.at[pl.ds(h * _CB, _CB)],
                            exp_hbm.at[pl.ds(h * _E + off, _CB)])
        return carry

    lax.fori_loop(0, _EPW // _CB, chunk_body, 0)
    pltpu.sync_copy(psum_v, psum_hbm.at[pl.ds(wid * _H * _N, _H * _N)])


def _edge_logits(src, dst, el1d, er1d):
    mesh = plsc.VectorSubcoreMesh(core_axis_name="c", subcore_axis_name="s")
    return pl.kernel(
        _edge_logits_body,
        compiler_params=pltpu.CompilerParams(needs_layout_passes=False, use_tc_tiling_on_sc=False),
        out_type=(
            jax.ShapeDtypeStruct((_H * _E,), jnp.float32),
            jax.ShapeDtypeStruct((_NW * _H * _N,), jnp.float32),
        ),
        mesh=mesh,
        scratch_types=[
            pltpu.VMEM((_H * _N,), jnp.float32),
            pltpu.VMEM((_H * _N,), jnp.float32),
            pltpu.VMEM((_H * _N,), jnp.float32),
            pltpu.VMEM((_CB,), jnp.int32),
            pltpu.VMEM((_CB,), jnp.int32),
            pltpu.VMEM((_H * _CB,), jnp.float32),
        ],
    )(src, dst, el1d, er1d)


def _recip_body(psum_ref, recip_ref):
    s = jnp.sum(psum_ref[...], axis=0)
    recip_ref[...] = 1.0 / s


def _denominators(psum):
    return pl.pallas_call(
        _recip_body,
        out_shape=jax.ShapeDtypeStruct((_H * _N,), jnp.float32),
    )(psum.reshape(_NW, _H * _N))


def _aggregate_body(h00, h01, h10, h11, h20, h21, exp_hbm, recip_hbm,
                    src3d_hbm, dst3d_hbm,
                    out_hbm,
                    src2d, dst2d, expv, recipv, attv,
                    r0, r1, f0, f1, acc,
                    g0, g1):
    cid = lax.axis_index("c")
    sid = lax.axis_index("s")
    wid = cid * 16 + sid
    base = pl.multiple_of(wid * _EPW, 8)

    pltpu.sync_copy(src3d_hbm.at[wid], src2d)
    pltpu.sync_copy(dst3d_hbm.at[wid], dst2d)

    zeros16 = jnp.zeros((16,), jnp.float32)
    h_tables = ((h00, h01), (h10, h11), (h20, h21))

    def zero_rows0(i, carry):
        for v in range(_HF // 16):
            f0[i, pl.ds(v * 16, 16)] = zeros16
        return carry

    def att_body(c, carry):
        for k in range(_CC // 16):
            d16 = dst2d[c, pl.ds(k * 16, 16)]
            r = plsc.load_gather(recipv, [d16])
            j = c * _CC + k * 16
            attv[pl.ds(j, 16)] = expv[pl.ds(j, 16)] * r
        return carry

    def scale(buf, out, att_base):
        def blk_body(j, carry):
            jbase = att_base + j * 16
            for i in range(16):
                idx = jnp.full((16,), i, jnp.int32) + jbase
                ab = plsc.load_gather(attv, [idx])
                row = j * 16 + i
                for g in range(_HF // 32):
                    v32 = buf[row, pl.ds(g * 32, 32)]
                    lo, hi = plsc.unpack(v32, format=plsc.PackFormat.INTERLEAVED)
                    out[row, pl.ds(g * 32, 16)] = lo * ab
                    out[row, pl.ds(g * 32 + 16, 16)] = hi * ab
            return carry
        lax.fori_loop(0, _CC // 16, blk_body, 0)

    for h in range(_H):
        pltpu.sync_copy(recip_hbm.at[pl.ds(h * _N, _N)], recipv)
        pltpu.sync_copy(exp_hbm.at[pl.ds(h * _E + base, _EPW)], expv)
        lax.fori_loop(0, _NCH, att_body, 0)

        for half in range(2):
            h_hbm = h_tables[h][half]

            lax.fori_loop(0, _CC, zero_rows0, 0)
            own = sid * _OWN
            for k in range(_OWN // _CC):
                pltpu.sync_copy(f0, acc.at[pl.ds(own + k * _CC, _CC)])
            rem = _OWN % _CC
            if rem:
                pltpu.sync_copy(f0.at[pl.ds(0, rem)],
                                acc.at[pl.ds(own + (_OWN // _CC) * _CC, rem)])

            @pl.when(sid == 15)
            def _():
                pltpu.sync_copy(f0.at[pl.ds(0, _TAIL)],
                                acc.at[pl.ds(16 * _OWN, _TAIL)])

            plsc.subcore_barrier()

            pltpu.async_copy(h_hbm.at[src2d.at[0]], r0, g0)

            def pair_body(p, carry):
                c0 = p * 2
                c1 = c0 + 1
                pltpu.async_copy(h_hbm.at[src2d.at[c1]], r1, g1)

                pltpu.make_async_copy(h_hbm.at[src2d.at[c0]], r0, g0).wait()
                scale(r0, f0, c0 * _CC)
                pltpu.async_copy(h_hbm.at[src2d.at[c0 + 2]], r0, g0)
                pltpu.sync_copy(f0, acc.at[dst2d.at[c0]], add=True)

                pltpu.make_async_copy(h_hbm.at[src2d.at[c1]], r1, g1).wait()
                scale(r1, f1, c1 * _CC)

                @pl.when(c1 + 2 < _NCH)
                def _():
                    pltpu.async_copy(h_hbm.at[src2d.at[c1 + 2]], r1, g1)

                pltpu.sync_copy(f1, acc.at[dst2d.at[c1]], add=True)
                return carry

            lax.fori_loop(0, (_NCH - 1) // 2, pair_body, 0)

            last = _NCH - 1
            pltpu.make_async_copy(h_hbm.at[src2d.at[last]], r0, g0).wait()
            scale(r0, f0, last * _CC)
            pltpu.sync_copy(f0, acc.at[dst2d.at[last]], add=True)

            plsc.subcore_barrier()

            slot = cid * 2 * _H + h * 2 + half
            pltpu.sync_copy(acc.at[pl.ds(own, _OWN)],
                            out_hbm.at[pl.ds(slot * _N + own, _OWN)])

            @pl.when(sid == 15)
            def _():
                pltpu.sync_copy(acc.at[pl.ds(16 * _OWN, _TAIL)],
                                out_hbm.at[pl.ds(slot * _N + 16 * _OWN, _TAIL)])

            plsc.subcore_barrier()


def _aggregate(halves, exp_e, recip, src3d, dst3d):
    mesh = plsc.VectorSubcoreMesh(core_axis_name="c", subcore_axis_name="s")
    return pl.kernel(
        _aggregate_body,
        compiler_params=pltpu.CompilerParams(needs_layout_passes=False, use_tc_tiling_on_sc=False),
        out_type=jax.ShapeDtypeStruct((2 * 2 * _H * _N, _HF), jnp.float32),
        mesh=mesh,
        scratch_types=[
            pltpu.VMEM((_NCH, _CC), jnp.int32),
            pltpu.VMEM((_NCH, _CC), jnp.int32),
            pltpu.VMEM((_EPW,), jnp.float32),
            pltpu.VMEM((_N,), jnp.float32),
            pltpu.VMEM((_EPW,), jnp.float32),
            pltpu.VMEM((_CC, _HF), jnp.bfloat16),
            pltpu.VMEM((_CC, _HF), jnp.bfloat16),
            pltpu.VMEM((_CC, _HF), jnp.float32),
            pltpu.VMEM((_CC, _HF), jnp.float32),
            pltpu.VMEM_SHARED((_N, _HF), jnp.float32),
            pltpu.SemaphoreType.DMA,
            pltpu.SemaphoreType.DMA,
        ],
    )(*halves, exp_e, recip, src3d, dst3d)


def _combine_body(part_ref, out_ref):
    p = part_ref[...]
    for h in range(_H):
        for half in range(2):
            s = h * 2 + half
            out_ref[:, h * _F + half * _HF:h * _F + (half + 1) * _HF] = (
                p[s] + p[2 * _H + s])


def _combine(part):
    return pl.pallas_call(
        _combine_body,
        grid=(_N // _BN,),
        in_specs=[pl.BlockSpec((4 * _H, _BN, _HF), lambda i: (0, i, 0))],
        out_specs=pl.BlockSpec((_BN, _H * _F), lambda i: (i, 0)),
        out_shape=jax.ShapeDtypeStruct((_N, _H * _F), jnp.float32),
    )(part.reshape(4 * _H, _N, _HF))


def kernel(x, edge_index, fc_weight, attn_l, attn_r):
    al = attn_l[0]
    ar = attn_r[0]
    eye = jnp.eye(_H, dtype=jnp.float32)
    a_l = (al[:, :, None] * eye[:, None, :]).reshape(_H * _F, _H)
    a_r = (ar[:, :, None] * eye[:, None, :]).reshape(_H * _F, _H)
    a8 = jnp.concatenate([a_l, a_r, jnp.zeros((_H * _F, 2), jnp.float32)], axis=1)
    wt = fc_weight.T

    g = jnp.arange(_H * _F, dtype=jnp.int32)
    base = (g // 32) * 32
    p = g % 32
    perm = base + (p % 2) * 16 + p // 2
    wt = wt[:, perm]
    a8 = a8[perm, :]

    src = edge_index[0]
    dst = edge_index[1]
    src3d = src.reshape(_NW, _NCH, _CC)
    dst3d = dst.reshape(_NW, _NCH, _CC)

    elr, *halves = _project(x, wt, a8)
    el1d = elr[:, 0:_H].reshape(-1)
    er1d = elr[:, _H:2 * _H].reshape(-1)

    exp_e, psum = _edge_logits(src, dst, el1d, er1d)
    recip = _denominators(psum)
    part = _aggregate(halves, exp_e, recip, src3d, dst3d)
    return _combine(part)


# speedup vs baseline: 1.3312x; 1.3312x over previous
import functools

import jax
import jax.numpy as jnp
from jax import lax
from jax.experimental import pallas as pl
from jax.experimental.pallas import tpu as pltpu
from jax.experimental.pallas import tpu_sc as plsc

_N = 10000
_E = 320000
_F = 128
_HF = 64
_H = 3
_NEG = 0.2

_NW = 32
_EPW = _E // _NW
_CB = 2000
_CC = 80
_NCH = _EPW // _CC
_OWN = 624
_TAIL = _N - 16 * _OWN

_NBUF = 4

_BN = 1000


def _proj_body(x_ref, wt_ref, a8_ref, elr_ref, *h_refs):
    h_all = jnp.dot(x_ref[...], wt_ref[...], preferred_element_type=jnp.float32)
    for i in range(2 * _H):
        h_refs[i][...] = h_all[:, i * _HF:(i + 1) * _HF]
    elr_ref[...] = jnp.dot(h_all, a8_ref[...], preferred_element_type=jnp.float32)


def _project(x, wt, a8):
    hspec = pl.BlockSpec((_BN, _HF), lambda i: (i, 0))
    hshape = jax.ShapeDtypeStruct((_N, _HF), jnp.float32)
    return pl.pallas_call(
        _proj_body,
        grid=(_N // _BN,),
        in_specs=[
            pl.BlockSpec((_BN, _F), lambda i: (i, 0)),
            pl.BlockSpec((_F, _H * _F), lambda i: (0, 0)),
            pl.BlockSpec((_H * _F, 8), lambda i: (0, 0)),
        ],
        out_specs=(pl.BlockSpec((_BN, 8), lambda i: (i, 0)),) + (hspec,) * (2 * _H),
        out_shape=(jax.ShapeDtypeStruct((_N, 8), jnp.float32),) + (hshape,) * (2 * _H),
    )(x, wt, a8)


def _edge_logits_body(src_hbm, dst_hbm, el_hbm, er_hbm,
                      exp_hbm, psum_hbm,
                      el_v, er_v, psum_v, src_c, dst_c, exp_c):
    cid = lax.axis_index("c")
    sid = lax.axis_index("s")
    wid = cid * 16 + sid
    base = pl.multiple_of(wid * _EPW, 8)

    pltpu.sync_copy(el_hbm, el_v)
    pltpu.sync_copy(er_hbm, er_v)

    zeros16 = jnp.zeros((16,), jnp.float32)

    def zbody(i, carry):
        psum_v[pl.ds(i * 16, 16)] = zeros16
        return carry

    lax.fori_loop(0, (_H * _N) // 16, zbody, 0)

    def chunk_body(c, carry):
        off = pl.multiple_of(base + c * _CB, 8)
        pltpu.sync_copy(src_hbm.at[pl.ds(off, _CB)], src_c)
        pltpu.sync_copy(dst_hbm.at[pl.ds(off, _CB)], dst_c)

        def vec_body(j, inner):
            s16 = src_c[pl.ds(j * 16, 16)]
            d16 = dst_c[pl.ds(j * 16, 16)]
            s3 = s16 * 3
            d3 = d16 * 3
            for h in range(_H):
                e = (plsc.load_gather(el_v, [s3 + h])
                     + plsc.load_gather(er_v, [d3 + h]))
                e = jnp.where(e >= 0, e, _NEG * e)
                ex = jnp.exp(e)
                exp_c[pl.ds(h * _CB + j * 16, 16)] = ex
                plsc.addupdate_scatter(psum_v, [d16 + h * _N], ex)
            return inner

        lax.fori_loop(0, _CB // 16, vec_body, 0)
        for h in range(_H):
            pltpu.sync_copy(exp_c.at[pl.ds(h * _CB, _CB)],
                            exp_hbm.at[pl.ds(h * _E + off, _CB)])
        return carry

    lax.fori_loop(0, _EPW // _CB, chunk_body, 0)
    pltpu.sync_copy(psum_v, psum_hbm.at[pl.ds(wid * _H * _N, _H * _N)])


def _edge_logits(src, dst, el1d, er1d):
    mesh = plsc.VectorSubcoreMesh(core_axis_name="c", subcore_axis_name="s")
    return pl.kernel(
        _edge_logits_body,
        compiler_params=pltpu.CompilerParams(needs_layout_passes=False, use_tc_tiling_on_sc=False),
        out_type=(
            jax.ShapeDtypeStruct((_H * _E,), jnp.float32),
            jax.ShapeDtypeStruct((_NW * _H * _N,), jnp.float32),
        ),
        mesh=mesh,
        scratch_types=[
            pltpu.VMEM((_H * _N,), jnp.float32),
            pltpu.VMEM((_H * _N,), jnp.float32),
            pltpu.VMEM((_H * _N,), jnp.float32),
            pltpu.VMEM((_CB,), jnp.int32),
            pltpu.VMEM((_CB,), jnp.int32),
            pltpu.VMEM((_H * _CB,), jnp.float32),
        ],
    )(src, dst, el1d, er1d)


def _recip_body(psum_ref, recip_ref):
    s = jnp.sum(psum_ref[...], axis=0)
    recip_ref[...] = 1.0 / s


def _denominators(psum):
    return pl.pallas_call(
        _recip_body,
        out_shape=jax.ShapeDtypeStruct((_H * _N,), jnp.float32),
    )(psum.reshape(_NW, _H * _N))


def _aggregate_body(h00, h01, h10, h11, h20, h21, exp_hbm, recip_hbm,
                    src3d_hbm, dst3d_hbm,
                    out_hbm,
                    src2d, dst2d, expv, recipv, attv,
                    r0, r1, f0, f1, acc,
                    g0, g1):
    cid = lax.axis_index("c")
    sid = lax.axis_index("s")
    wid = cid * 16 + sid
    base = pl.multiple_of(wid * _EPW, 8)

    pltpu.sync_copy(src3d_hbm.at[wid], src2d)
    pltpu.sync_copy(dst3d_hbm.at[wid], dst2d)

    zeros16 = jnp.zeros((16,), jnp.float32)
    h_tables = ((h00, h01), (h10, h11), (h20, h21))

    def zero_rows0(i, carry):
        for v in range(_HF // 16):
            f0[i, pl.ds(v * 16, 16)] = zeros16
        return carry

    def att_body(c, carry):
        for k in range(_CC // 16):
            d16 = dst2d[c, pl.ds(k * 16, 16)]
            r = plsc.load_gather(recipv, [d16])
            j = c * _CC + k * 16
            attv[pl.ds(j, 16)] = expv[pl.ds(j, 16)] * r
        return carry

    def scale(buf, out, att_base):
        def blk_body(j, carry):
            jbase = att_base + j * 16
            for i in range(16):
                idx = jnp.full((16,), i, jnp.int32) + jbase
                ab = plsc.load_gather(attv, [idx])
                row = j * 16 + i
                for v in range(_HF // 16):
                    sl = pl.ds(v * 16, 16)
                    out[row, sl] = buf[row, sl] * ab
            return carry
        lax.fori_loop(0, _CC // 16, blk_body, 0)

    for h in range(_H):
        pltpu.sync_copy(recip_hbm.at[pl.ds(h * _N, _N)], recipv)
        pltpu.sync_copy(exp_hbm.at[pl.ds(h * _E + base, _EPW)], expv)
        lax.fori_loop(0, _NCH, att_body, 0)

        for half in range(2):
            h_hbm = h_tables[h][half]

            lax.fori_loop(0, _CC, zero_rows0, 0)
            own = sid * _OWN
            for k in range(_OWN // _CC):
                pltpu.sync_copy(f0, acc.at[pl.ds(own + k * _CC, _CC)])
            rem = _OWN % _CC
            if rem:
                pltpu.sync_copy(f0.at[pl.ds(0, rem)],
                                acc.at[pl.ds(own + (_OWN // _CC) * _CC, rem)])

            @pl.when(sid == 15)
            def _():
                pltpu.sync_copy(f0.at[pl.ds(0, _TAIL)],
                                acc.at[pl.ds(16 * _OWN, _TAIL)])

            plsc.subcore_barrier()

            pltpu.async_copy(h_hbm.at[src2d.at[0]], r0, g0)

            def pair_body(p, carry):
                c0 = p * 2
                c1 = c0 + 1
                pltpu.async_copy(h_hbm.at[src2d.at[c1]], r1, g1)

                pltpu.make_async_copy(h_hbm.at[src2d.at[c0]], r0, g0).wait()
                scale(r0, f0, c0 * _CC)
                pltpu.async_copy(h_hbm.at[src2d.at[c0 + 2]], r0, g0)
                pltpu.sync_copy(f0, acc.at[dst2d.at[c0]], add=True)

                pltpu.make_async_copy(h_hbm.at[src2d.at[c1]], r1, g1).wait()
                scale(r1, f1, c1 * _CC)

                @pl.when(c1 + 2 < _NCH)
                def _():
                    pltpu.async_copy(h_hbm.at[src2d.at[c1 + 2]], r1, g1)

                pltpu.sync_copy(f1, acc.at[dst2d.at[c1]], add=True)
                return carry

            lax.fori_loop(0, (_NCH - 1) // 2, pair_body, 0)

            last = _NCH - 1
            pltpu.make_async_copy(h_hbm.at[src2d.at[last]], r0, g0).wait()
            scale(r0, f0, last * _CC)
            pltpu.sync_copy(f0, acc.at[dst2d.at[last]], add=True)

            plsc.subcore_barrier()

            slot = cid * 2 * _H + h * 2 + half
            pltpu.sync_copy(acc.at[pl.ds(own, _OWN)],
                            out_hbm.at[pl.ds(slot * _N + own, _OWN)])

            @pl.when(sid == 15)
            def _():
                pltpu.sync_copy(acc.at[pl.ds(16 * _OWN, _TAIL)],
                                out_hbm.at[pl.ds(slot * _N + 16 * _OWN, _TAIL)])

            plsc.subcore_barrier()


def _aggregate(halves, exp_e, recip, src3d, dst3d):
    mesh = plsc.VectorSubcoreMesh(core_axis_name="c", subcore_axis_name="s")
    return pl.kernel(
        _aggregate_body,
        compiler_params=pltpu.CompilerParams(needs_layout_passes=False, use_tc_tiling_on_sc=False),
        out_type=jax.ShapeDtypeStruct((2 * 2 * _H * _N, _HF), jnp.float32),
        mesh=mesh,
        scratch_types=[
            pltpu.VMEM((_NCH, _CC), jnp.int32),
            pltpu.VMEM((_NCH, _CC), jnp.int32),
            pltpu.VMEM((_EPW,), jnp.float32),
            pltpu.VMEM((_N,), jnp.float32),
            pltpu.VMEM((_EPW,), jnp.float32),
            pltpu.VMEM((_CC, _HF), jnp.float32),
            pltpu.VMEM((_CC, _HF), jnp.float32),
            pltpu.VMEM((_CC, _HF), jnp.float32),
            pltpu.VMEM((_CC, _HF), jnp.float32),
            pltpu.VMEM_SHARED((_N, _HF), jnp.float32),
            pltpu.SemaphoreType.DMA,
            pltpu.SemaphoreType.DMA,
        ],
    )(*halves, exp_e, recip, src3d, dst3d)


def _combine_body(part_ref, out_ref):
    p = part_ref[...]
    for h in range(_H):
        for half in range(2):
            s = h * 2 + half
            out_ref[:, h * _F + half * _HF:h * _F + (half + 1) * _HF] = (
                p[s] + p[2 * _H + s])


def _combine(part):
    return pl.pallas_call(
        _combine_body,
        grid=(_N // _BN,),
        in_specs=[pl.BlockSpec((4 * _H, _BN, _HF), lambda i: (0, i, 0))],
        out_specs=pl.BlockSpec((_BN, _H * _F), lambda i: (i, 0)),
        out_shape=jax.ShapeDtypeStruct((_N, _H * _F), jnp.float32),
    )(part.reshape(4 * _H, _N, _HF))


def kernel(x, edge_index, fc_weight, attn_l, attn_r):
    al = attn_l[0]
    ar = attn_r[0]
    eye = jnp.eye(_H, dtype=jnp.float32)
    a_l = (al[:, :, None] * eye[:, None, :]).reshape(_H * _F, _H)
    a_r = (ar[:, :, None] * eye[:, None, :]).reshape(_H * _F, _H)
    a8 = jnp.concatenate([a_l, a_r, jnp.zeros((_H * _F, 2), jnp.float32)], axis=1)
    wt = fc_weight.T

    src = edge_index[0]
    dst = edge_index[1]
    src3d = src.reshape(_NW, _NCH, _CC)
    dst3d = dst.reshape(_NW, _NCH, _CC)

    elr, *halves = _project(x, wt, a8)
    el1d = elr[:, 0:_H].reshape(-1)
    er1d = elr[:, _H:2 * _H].reshape(-1)

    exp_e, psum = _edge_logits(src, dst, el1d, er1d)
    recip = _denominators(psum)
    part = _aggregate(halves, exp_e, recip, src3d, dst3d)
    return _combine(part)


# speedup vs baseline: 1.4308x; 1.0748x over previous
import functools

import jax
import jax.numpy as jnp
from jax import lax
from jax.experimental import pallas as pl
from jax.experimental.pallas import tpu as pltpu
from jax.experimental.pallas import tpu_sc as plsc

_N = 10000
_E = 320000
_F = 128
_HF = 64
_H = 3
_NEG = 0.2

_NW = 32
_EPW = _E // _NW
_CB = 2000
_CC = 80
_NCH = _EPW // _CC
_OWN = 624
_TAIL = _N - 16 * _OWN

_NBUF = 4

_BN = 1000


def _proj_body(x_ref, wt_ref, a8_ref, elr_ref, *h_refs):
    h_all = jnp.dot(x_ref[...], wt_ref[...], preferred_element_type=jnp.float32)
    for i in range(2 * _H):
        h_refs[i][...] = h_all[:, i * _HF:(i + 1) * _HF]
    elr_ref[...] = jnp.dot(h_all, a8_ref[...], preferred_element_type=jnp.float32)


def _project(x, wt, a8):
    hspec = pl.BlockSpec((_BN, _HF), lambda i: (i, 0))
    hshape = jax.ShapeDtypeStruct((_N, _HF), jnp.float32)
    return pl.pallas_call(
        _proj_body,
        grid=(_N // _BN,),
        in_specs=[
            pl.BlockSpec((_BN, _F), lambda i: (i, 0)),
            pl.BlockSpec((_F, _H * _F), lambda i: (0, 0)),
            pl.BlockSpec((_H * _F, 8), lambda i: (0, 0)),
        ],
        out_specs=(pl.BlockSpec((_BN, 8), lambda i: (i, 0)),) + (hspec,) * (2 * _H),
        out_shape=(jax.ShapeDtypeStruct((_N, 8), jnp.float32),) + (hshape,) * (2 * _H),
    )(x, wt, a8)


def _edge_logits_body(src_hbm, dst_hbm, el_hbm, er_hbm,
                      exp_hbm, psum_hbm,
                      el_v, er_v, psum_v, src_c, dst_c, exp_c):
    cid = lax.axis_index("c")
    sid = lax.axis_index("s")
    wid = cid * 16 + sid
    base = pl.multiple_of(wid * _EPW, 8)

    pltpu.sync_copy(el_hbm, el_v)
    pltpu.sync_copy(er_hbm, er_v)

    zeros16 = jnp.zeros((16,), jnp.float32)

    def zbody(i, carry):
        psum_v[pl.ds(i * 16, 16)] = zeros16
        return carry

    lax.fori_loop(0, (_H * _N) // 16, zbody, 0)

    def chunk_body(c, carry):
        off = pl.multiple_of(base + c * _CB, 8)
        pltpu.sync_copy(src_hbm.at[pl.ds(off, _CB)], src_c)
        pltpu.sync_copy(dst_hbm.at[pl.ds(off, _CB)], dst_c)

        def vec_body(j, inner):
            s16 = src_c[pl.ds(j * 16, 16)]
            d16 = dst_c[pl.ds(j * 16, 16)]
            s3 = s16 * 3
            d3 = d16 * 3
            for h in range(_H):
                e = (plsc.load_gather(el_v, [s3 + h])
                     + plsc.load_gather(er_v, [d3 + h]))
                e = jnp.where(e >= 0, e, _NEG * e)
                ex = jnp.exp(e)
                exp_c[pl.ds(h * _CB + j * 16, 16)] = ex
                plsc.addupdate_scatter(psum_v, [d16 + h * _N], ex)
            return inner

        lax.fori_loop(0, _CB // 16, vec_body, 0)
        for h in range(_H):
            pltpu.sync_copy(exp_c.at[pl.ds(h * _CB, _CB)],
                            exp_hbm.at[pl.ds(h * _E + off, _CB)])
        return carry

    lax.fori_loop(0, _EPW // _CB, chunk_body, 0)
    pltpu.sync_copy(psum_v, psum_hbm.at[pl.ds(wid * _H * _N, _H * _N)])


def _edge_logits(src, dst, el1d, er1d):
    mesh = plsc.VectorSubcoreMesh(core_axis_name="c", subcore_axis_name="s")
    return pl.kernel(
        _edge_logits_body,
        compiler_params=pltpu.CompilerParams(needs_layout_passes=False, use_tc_tiling_on_sc=False),
        out_type=(
            jax.ShapeDtypeStruct((_H * _E,), jnp.float32),
            jax.ShapeDtypeStruct((_NW * _H * _N,), jnp.float32),
        ),
        mesh=mesh,
        scratch_types=[
            pltpu.VMEM((_H * _N,), jnp.float32),
            pltpu.VMEM((_H * _N,), jnp.float32),
            pltpu.VMEM((_H * _N,), jnp.float32),
            pltpu.VMEM((_CB,), jnp.int32),
            pltpu.VMEM((_CB,), jnp.int32),
            pltpu.VMEM((_H * _CB,), jnp.float32),
        ],
    )(src, dst, el1d, er1d)


def _recip_body(psum_ref, recip_ref):
    s = jnp.sum(psum_ref[...], axis=0)
    recip_ref[...] = 1.0 / s


def _denominators(psum):
    return pl.pallas_call(
        _recip_body,
        out_shape=jax.ShapeDtypeStruct((_H * _N,), jnp.float32),
    )(psum.reshape(_NW, _H * _N))


def _aggregate_body(h00, h01, h10, h11, h20, h21, exp_hbm, recip_hbm,
                    src3d_hbm, dst3d_hbm,
                    out_hbm,
                    src2d, dst2d, expv, recipv, attv,
                    r0, r1, f0, f1, acc,
                    g0, g1, s0, s1):
    cid = lax.axis_index("c")
    sid = lax.axis_index("s")
    wid = cid * 16 + sid
    base = pl.multiple_of(wid * _EPW, 8)

    pltpu.sync_copy(src3d_hbm.at[wid], src2d)
    pltpu.sync_copy(dst3d_hbm.at[wid], dst2d)

    zeros16 = jnp.zeros((16,), jnp.float32)
    h_tables = ((h00, h01), (h10, h11), (h20, h21))

    def zero_rows0(i, carry):
        for v in range(_HF // 16):
            f0[i, pl.ds(v * 16, 16)] = zeros16
        return carry

    def att_body(c, carry):
        for k in range(_CC // 16):
            d16 = dst2d[c, pl.ds(k * 16, 16)]
            r = plsc.load_gather(recipv, [d16])
            j = c * _CC + k * 16
            attv[pl.ds(j, 16)] = expv[pl.ds(j, 16)] * r
        return carry

    def scale(buf, out, att_base):
        def blk_body(j, carry):
            jbase = att_base + j * 16
            for i in range(16):
                idx = jnp.full((16,), i, jnp.int32) + jbase
                ab = plsc.load_gather(attv, [idx])
                row = j * 16 + i
                for v in range(_HF // 16):
                    sl = pl.ds(v * 16, 16)
                    out[row, sl] = buf[row, sl] * ab
            return carry
        lax.fori_loop(0, _CC // 16, blk_body, 0)

    for h in range(_H):
        pltpu.sync_copy(recip_hbm.at[pl.ds(h * _N, _N)], recipv)
        pltpu.sync_copy(exp_hbm.at[pl.ds(h * _E + base, _EPW)], expv)
        lax.fori_loop(0, _NCH, att_body, 0)

        for half in range(2):
            h_hbm = h_tables[h][half]

            lax.fori_loop(0, _CC, zero_rows0, 0)
            own = sid * _OWN
            for k in range(_OWN // _CC):
                pltpu.sync_copy(f0, acc.at[pl.ds(own + k * _CC, _CC)])
            rem = _OWN % _CC
            if rem:
                pltpu.sync_copy(f0.at[pl.ds(0, rem)],
                                acc.at[pl.ds(own + (_OWN // _CC) * _CC, rem)])

            @pl.when(sid == 15)
            def _():
                pltpu.sync_copy(f0.at[pl.ds(0, _TAIL)],
                                acc.at[pl.ds(16 * _OWN, _TAIL)])

            plsc.subcore_barrier()

            pltpu.async_copy(h_hbm.at[src2d.at[0]], r0, g0)

            def pair_body(p, carry):
                c0 = p * 2
                c1 = c0 + 1
                pltpu.async_copy(h_hbm.at[src2d.at[c1]], r1, g1)

                pltpu.make_async_copy(h_hbm.at[src2d.at[c0]], r0, g0).wait()

                @pl.when(p > 0)
                def _():
                    pltpu.make_async_copy(f0, acc.at[dst2d.at[c0 - 2]],
                                          s0).wait()

                scale(r0, f0, c0 * _CC)
                pltpu.async_copy(h_hbm.at[src2d.at[c0 + 2]], r0, g0)
                pltpu.async_copy(f0, acc.at[dst2d.at[c0]], s0, add=True)

                pltpu.make_async_copy(h_hbm.at[src2d.at[c1]], r1, g1).wait()

                @pl.when(p > 0)
                def _():
                    pltpu.make_async_copy(f1, acc.at[dst2d.at[c1 - 2]],
                                          s1).wait()

                scale(r1, f1, c1 * _CC)

                @pl.when(c1 + 2 < _NCH)
                def _():
                    pltpu.async_copy(h_hbm.at[src2d.at[c1 + 2]], r1, g1)

                pltpu.async_copy(f1, acc.at[dst2d.at[c1]], s1, add=True)
                return carry

            lax.fori_loop(0, (_NCH - 1) // 2, pair_body, 0)

            last = _NCH - 1
            pltpu.make_async_copy(h_hbm.at[src2d.at[last]], r0, g0).wait()
            pltpu.make_async_copy(f0, acc.at[dst2d.at[last - 2]], s0).wait()
            scale(r0, f0, last * _CC)
            pltpu.sync_copy(f0, acc.at[dst2d.at[last]], add=True)
            pltpu.make_async_copy(f1, acc.at[dst2d.at[last - 1]], s1).wait()

            plsc.subcore_barrier()

            slot = cid * 2 * _H + h * 2 + half
            pltpu.sync_copy(acc.at[pl.ds(own, _OWN)],
                            out_hbm.at[pl.ds(slot * _N + own, _OWN)])

            @pl.when(sid == 15)
            def _():
                pltpu.sync_copy(acc.at[pl.ds(16 * _OWN, _TAIL)],
                                out_hbm.at[pl.ds(slot * _N + 16 * _OWN, _TAIL)])

            plsc.subcore_barrier()


def _aggregate(halves, exp_e, recip, src3d, dst3d):
    mesh = plsc.VectorSubcoreMesh(core_axis_name="c", subcore_axis_name="s")
    return pl.kernel(
        _aggregate_body,
        compiler_params=pltpu.CompilerParams(needs_layout_passes=False, use_tc_tiling_on_sc=False),
        out_type=jax.ShapeDtypeStruct((2 * 2 * _H * _N, _HF), jnp.float32),
        mesh=mesh,
        scratch_types=[
            pltpu.VMEM((_NCH, _CC), jnp.int32),
            pltpu.VMEM((_NCH, _CC), jnp.int32),
            pltpu.VMEM((_EPW,), jnp.float32),
            pltpu.VMEM((_N,), jnp.float32),
            pltpu.VMEM((_EPW,), jnp.float32),
            pltpu.VMEM((_CC, _HF), jnp.float32),
            pltpu.VMEM((_CC, _HF), jnp.float32),
            pltpu.VMEM((_CC, _HF), jnp.float32),
            pltpu.VMEM((_CC, _HF), jnp.float32),
            pltpu.VMEM_SHARED((_N, _HF), jnp.float32),
            pltpu.SemaphoreType.DMA,
            pltpu.SemaphoreType.DMA,
            pltpu.SemaphoreType.DMA,
            pltpu.SemaphoreType.DMA,
        ],
    )(*halves, exp_e, recip, src3d, dst3d)


def _combine_body(part_ref, out_ref):
    p = part_ref[...]
    for h in range(_H):
        for half in range(2):
            s = h * 2 + half
            out_ref[:, h * _F + half * _HF:h * _F + (half + 1) * _HF] = (
                p[s] + p[2 * _H + s])


def _combine(part):
    return pl.pallas_call(
        _combine_body,
        grid=(_N // _BN,),
        in_specs=[pl.BlockSpec((4 * _H, _BN, _HF), lambda i: (0, i, 0))],
        out_specs=pl.BlockSpec((_BN, _H * _F), lambda i: (i, 0)),
        out_shape=jax.ShapeDtypeStruct((_N, _H * _F), jnp.float32),
    )(part.reshape(4 * _H, _N, _HF))


def kernel(x, edge_index, fc_weight, attn_l, attn_r):
    al = attn_l[0]
    ar = attn_r[0]
    eye = jnp.eye(_H, dtype=jnp.float32)
    a_l = (al[:, :, None] * eye[:, None, :]).reshape(_H * _F, _H)
    a_r = (ar[:, :, None] * eye[:, None, :]).reshape(_H * _F, _H)
    a8 = jnp.concatenate([a_l, a_r, jnp.zeros((_H * _F, 2), jnp.float32)], axis=1)
    wt = fc_weight.T

    src = edge_index[0]
    dst = edge_index[1]
    src3d = src.reshape(_NW, _NCH, _CC)
    dst3d = dst.reshape(_NW, _NCH, _CC)

    elr, *halves = _project(x, wt, a8)
    el1d = elr[:, 0:_H].reshape(-1)
    er1d = elr[:, _H:2 * _H].reshape(-1)

    exp_e, psum = _edge_logits(src, dst, el1d, er1d)
    recip = _denominators(psum)
    part = _aggregate(halves, exp_e, recip, src3d, dst3d)
    return _combine(part)
